# baseline (speedup 1.0000x reference)
import functools

import jax
import jax.numpy as jnp
from jax import lax
from jax.experimental import pallas as pl
from jax.experimental.pallas import tpu as pltpu
from jax.experimental.pallas import tpu_sc as plsc

_NUM_NODES = 1000000
_EMBED_DIM = 64
_BATCH = 16384

_NC = 2
_NS = 16
_NW = _NC * _NS
_L = 16
_G = 128
_NG = -(-_NUM_NODES // _G)
_GQ = _NG // _NW
_GR = _NG % _NW
_CAP = 512

_mesh = plsc.VectorSubcoreMesh(core_axis_name="c", subcore_axis_name="s")


@functools.partial(
    pl.kernel,
    mesh=_mesh,
    out_type=jax.ShapeDtypeStruct((_BATCH * _EMBED_DIM,), jnp.float32),
    compiler_params=pltpu.CompilerParams(needs_layout_passes=False),
    scratch_types=[
        pltpu.VMEM((_BATCH,), jnp.int32),
        pltpu.VMEM((_BATCH + _L,), jnp.int32),
        pltpu.VMEM((_BATCH + _L,), jnp.int32),
        pltpu.VMEM((2, _EMBED_DIM, _G), jnp.float32),
        pltpu.VMEM((_CAP * _EMBED_DIM,), jnp.float32),
        pltpu.SMEM((4,), jnp.int32),
        pltpu.SemaphoreType.DMA,
        pltpu.SemaphoreType.DMA,
        pltpu.SemaphoreType.DMA,
    ],
)
def _gather_kernel(idx_hbm, table_hbm, out_hbm, gidx_v, nlist, plist,
                   stage, rows_v, cnt_sm, sem_a, sem_b, sem_r):
    wid = lax.axis_index("s") * _NC + lax.axis_index("c")
    g0 = wid * _GQ + lax.min(wid, _GR)
    cntg = _GQ + jnp.where(wid < _GR, 1, 0)

    pltpu.sync_copy(idx_hbm, gidx_v)
    cnt_sm[0] = 0
    cnt_sm[1] = 0

    lo = jnp.full((_L,), g0 * _G, jnp.int32)
    hi = jnp.full((_L,), (g0 + cntg) * _G, jnp.int32)
    iota = lax.iota(jnp.int32, _L)

    @pl.loop(0, _BATCH // _L)
    def _scan(i):
        v = gidx_v[pl.ds(i * _L, _L)]
        m = jnp.logical_and(v >= lo, v < hi)
        pc = plsc.all_reduce_population_count(m)
        off = cnt_sm[0]
        plsc.store_compressed(nlist.at[pl.ds(off, _L)], v, mask=m)
        plsc.store_compressed(plist.at[pl.ds(off, _L)], iota + i * _L, mask=m)
        cnt_sm[0] = off + pc[0]

    tot = cnt_sm[0]
    nlist[pl.ds(tot, _L)] = jnp.full((_L,), -1, jnp.int32)

    nscan = lax.shift_right_logical(tot + _L - 1, 4)
    kvecs = [iota + (g * _L) for g in range(_EMBED_DIM // _L)]

    def fire(q, buf, sem):
        off = pl.multiple_of(q * _G, _G)
        pltpu.async_copy(table_hbm.at[:, pl.ds(off, _G)], stage.at[buf], sem)

    def drain(buf, sem):
        pltpu.make_async_copy(
            table_hbm.at[:, pl.ds(0, _G)], stage.at[buf], sem
        ).wait()

    def scan_extract(qcur, buf):
        qv = jnp.full((_L,), qcur, jnp.int32)

        @pl.loop(0, nscan)
        def _s(s):
            lv = nlist[pl.ds(s * _L, _L)]
            m = jnp.right_shift(lv, 7) == qv
            pc = plsc.all_reduce_population_count(m)

            @pl.when(pc[0] > 0)
            def _():
                pv = plist[pl.ds(s * _L, _L)]
                cv = jnp.bitwise_and(lv, _G - 1)
                mi = jnp.where(m, 1, 0)
                for k in range(_L):
                    @pl.when(mi[k] == 1)
                    def _():
                        c = cv[k]
                        pos = pv[k]
                        rs = cnt_sm[1]
                        slot = jnp.bitwise_and(rs, _CAP - 1)
                        cvec = jnp.full((_L,), c, jnp.int32)
                        for kg, kvec in enumerate(kvecs):
                            vals = plsc.load_gather(stage.at[buf], [kvec, cvec])
                            rows_v[pl.ds(slot * _EMBED_DIM + kg * _L, _L)] = vals
                        pltpu.async_copy(
                            rows_v.at[pl.ds(slot * _EMBED_DIM, _EMBED_DIM)],
                            out_hbm.at[pl.ds(pos * _EMBED_DIM, _EMBED_DIM)],
                            sem_r,
                        )
                        cnt_sm[1] = rs + 1

    fire(g0, 0, sem_a)

    @pl.when(cntg > 1)
    def _():
        fire(g0 + 1, 1, sem_b)

    @pl.loop(0, lax.shift_right_logical(cntg + 1, 1))
    def _pair(u):
        qa = g0 + 2 * u
        drain(0, sem_a)
        scan_extract(qa, 0)

        @pl.when(2 * u + 2 < cntg)
        def _():
            fire(qa + 2, 0, sem_a)

        @pl.when(2 * u + 1 < cntg)
        def _():
            drain(1, sem_b)
            scan_extract(qa + 1, 1)

            @pl.when(2 * u + 3 < cntg)
            def _():
                fire(qa + 3, 1, sem_b)

    nrows = cnt_sm[1]

    @pl.loop(0, nrows)
    def _dr(u):
        pltpu.make_async_copy(
            rows_v.at[pl.ds(0, _EMBED_DIM)],
            out_hbm.at[pl.ds(0, _EMBED_DIM)],
            sem_r,
        ).wait()


def kernel(batch, table):
    flat = _gather_kernel(batch, table.T)
    return flat.reshape(_BATCH, _EMBED_DIM)


# speedup vs baseline: 1.3626x; 1.3626x over previous
import functools

import jax
import jax.numpy as jnp
from jax import lax
from jax.experimental import pallas as pl
from jax.experimental.pallas import tpu as pltpu
from jax.experimental.pallas import tpu_sc as plsc

_NUM_NODES = 1000000
_EMBED_DIM = 64
_BATCH = 16384

_NC = 2
_NS = 16
_NW = _NC * _NS
_L = 16
_G = 128
_W = 512
_NSLAB = -(-_NUM_NODES // _W)
_SQ = _NSLAB // _NW
_SR = _NSLAB % _NW
_CAP = 128

_mesh = plsc.VectorSubcoreMesh(core_axis_name="c", subcore_axis_name="s")


@functools.partial(
    pl.kernel,
    mesh=_mesh,
    out_type=jax.ShapeDtypeStruct((_BATCH * _EMBED_DIM,), jnp.float32),
    compiler_params=pltpu.CompilerParams(needs_layout_passes=False),
    scratch_types=[
        pltpu.VMEM((_BATCH,), jnp.int32),
        pltpu.VMEM((_BATCH + _L,), jnp.int32),
        pltpu.VMEM((_BATCH + _L,), jnp.int32),
        pltpu.VMEM((2, _EMBED_DIM, _W), jnp.float32),
        pltpu.VMEM((_CAP * _EMBED_DIM,), jnp.float32),
        pltpu.SMEM((4,), jnp.int32),
        pltpu.SemaphoreType.DMA,
        pltpu.SemaphoreType.DMA,
        pltpu.SemaphoreType.DMA,
    ],
)
def _gather_kernel(idx_hbm, table_hbm, out_hbm, gidx_v, nlist, plist,
                   stage, rows_v, cnt_sm, sem_a, sem_b, sem_r):
    wid = lax.axis_index("s") * _NC + lax.axis_index("c")
    s0 = wid * _SQ + lax.min(wid, _SR)
    cnts = _SQ + jnp.where(wid < _SR, 1, 0)

    pltpu.sync_copy(idx_hbm, gidx_v)
    cnt_sm[0] = 0
    cnt_sm[1] = 0

    lo = jnp.full((_L,), s0 * _W, jnp.int32)
    hi = jnp.full((_L,), (s0 + cnts) * _W, jnp.int32)
    iota = lax.iota(jnp.int32, _L)

    @pl.loop(0, _BATCH // _L)
    def _scan(i):
        v = gidx_v[pl.ds(i * _L, _L)]
        m = jnp.logical_and(v >= lo, v < hi)
        pc = plsc.all_reduce_population_count(m)
        off = cnt_sm[0]
        plsc.store_compressed(nlist.at[pl.ds(off, _L)], v, mask=m)
        plsc.store_compressed(plist.at[pl.ds(off, _L)], iota + i * _L, mask=m)
        cnt_sm[0] = off + pc[0]

    tot = cnt_sm[0]
    nlist[pl.ds(tot, _L)] = jnp.full((_L,), -1, jnp.int32)

    nscan = lax.shift_right_logical(tot + _L - 1, 4)
    kvecs = [iota + (g * _L) for g in range(_EMBED_DIM // _L)]

    def fire(q, buf, sem):
        @pl.when(q == _NSLAB - 1)
        def _():
            off = pl.multiple_of(q * _W, _G)
            pltpu.async_copy(
                table_hbm.at[:, pl.ds(off, _G)],
                stage.at[buf].at[:, pl.ds(0, _G)],
                sem,
            )

        @pl.when(q < _NSLAB - 1)
        def _():
            off = pl.multiple_of(q * _W, _W)
            pltpu.async_copy(
                table_hbm.at[:, pl.ds(off, _W)], stage.at[buf], sem
            )

    def drain(q, buf, sem):
        @pl.when(q == _NSLAB - 1)
        def _():
            pltpu.make_async_copy(
                table_hbm.at[:, pl.ds(0, _G)],
                stage.at[buf].at[:, pl.ds(0, _G)],
                sem,
            ).wait()

        @pl.when(q < _NSLAB - 1)
        def _():
            pltpu.make_async_copy(
                table_hbm.at[:, pl.ds(0, _W)], stage.at[buf], sem
            ).wait()

    def scan_extract(qcur, buf):
        qv = jnp.full((_L,), qcur, jnp.int32)

        @pl.loop(0, nscan)
        def _s(s):
            lv = nlist[pl.ds(s * _L, _L)]
            m = jnp.right_shift(lv, 9) == qv
            pc = plsc.all_reduce_population_count(m)

            @pl.when(pc[0] > 0)
            def _():
                pv = plist[pl.ds(s * _L, _L)]
                cv = jnp.bitwise_and(lv, _W - 1)
                mi = jnp.where(m, 1, 0)
                for k in range(_L):
                    @pl.when(mi[k] == 1)
                    def _():
                        c = cv[k]
                        pos = pv[k]
                        rs = cnt_sm[1]
                        slot = jnp.bitwise_and(rs, _CAP - 1)
                        cvec = jnp.full((_L,), c, jnp.int32)
                        for kg, kvec in enumerate(kvecs):
                            vals = plsc.load_gather(stage.at[buf], [kvec, cvec])
                            rows_v[pl.ds(slot * _EMBED_DIM + kg * _L, _L)] = vals
                        pltpu.async_copy(
                            rows_v.at[pl.ds(slot * _EMBED_DIM, _EMBED_DIM)],
                            out_hbm.at[pl.ds(pos * _EMBED_DIM, _EMBED_DIM)],
                            sem_r,
                        )
                        cnt_sm[1] = rs + 1

    fire(s0, 0, sem_a)

    @pl.when(cnts > 1)
    def _():
        fire(s0 + 1, 1, sem_b)

    @pl.loop(0, lax.shift_right_logical(cnts + 1, 1))
    def _pair(u):
        qa = s0 + 2 * u
        drain(qa, 0, sem_a)
        scan_extract(qa, 0)

        @pl.when(2 * u + 2 < cnts)
        def _():
            fire(qa + 2, 0, sem_a)

        @pl.when(2 * u + 1 < cnts)
        def _():
            drain(qa + 1, 1, sem_b)
            scan_extract(qa + 1, 1)

            @pl.when(2 * u + 3 < cnts)
            def _():
                fire(qa + 3, 1, sem_b)

    nrows = cnt_sm[1]

    @pl.loop(0, nrows)
    def _dr(u):
        pltpu.make_async_copy(
            rows_v.at[pl.ds(0, _EMBED_DIM)],
            out_hbm.at[pl.ds(0, _EMBED_DIM)],
            sem_r,
        ).wait()


def kernel(batch, table):
    flat = _gather_kernel(batch, table.T)
    return flat.reshape(_BATCH, _EMBED_DIM)


# speedup vs baseline: 2.3919x; 1.7554x over previous
import functools

import jax
import jax.numpy as jnp
from jax import lax
from jax.experimental import pallas as pl
from jax.experimental.pallas import tpu as pltpu
from jax.experimental.pallas import tpu_sc as plsc

_NUM_NODES = 1000000
_EMBED_DIM = 64
_BATCH = 16384

_NC = 2
_NS = 16
_NW = _NC * _NS
_L = 16
_G = 128
_W = 512
_NSLAB = -(-_NUM_NODES // _W)
_SQ = _NSLAB // _NW
_SR = _NSLAB % _NW
_CAP = 128

_mesh = plsc.VectorSubcoreMesh(core_axis_name="c", subcore_axis_name="s")


@functools.partial(
    pl.kernel,
    mesh=_mesh,
    out_type=jax.ShapeDtypeStruct((_BATCH * _EMBED_DIM,), jnp.float32),
    compiler_params=pltpu.CompilerParams(needs_layout_passes=False),
    scratch_types=[
        pltpu.VMEM((_BATCH + _L,), jnp.int32),
        pltpu.VMEM((_BATCH + _L,), jnp.int32),
        pltpu.VMEM((_BATCH + _L,), jnp.int32),
        pltpu.VMEM((2, _EMBED_DIM, _W), jnp.float32),
        pltpu.VMEM((_CAP * _EMBED_DIM,), jnp.float32),
        pltpu.SMEM((64,), jnp.int32),
        pltpu.SemaphoreType.DMA,
        pltpu.SemaphoreType.DMA,
        pltpu.SemaphoreType.DMA,
    ],
)
def _gather_kernel(idx_hbm, table_hbm, out_hbm, gidx_v, nlist, plist,
                   stage, rows_v, cnt_sm, sem_a, sem_b, sem_r):
    wid = lax.axis_index("s") * _NC + lax.axis_index("c")
    s0 = wid * _SQ + lax.min(wid, _SR)
    cnts = _SQ + jnp.where(wid < _SR, 1, 0)

    pltpu.sync_copy(idx_hbm, gidx_v.at[pl.ds(0, _BATCH)])
    cnt_sm[0] = 0
    cnt_sm[1] = 0

    lo = jnp.full((_L,), s0 * _W, jnp.int32)
    hi = jnp.full((_L,), (s0 + cnts) * _W, jnp.int32)
    iota = lax.iota(jnp.int32, _L)

    @pl.loop(0, _BATCH // _L)
    def _scan(i):
        v = gidx_v[pl.ds(i * _L, _L)]
        m = jnp.logical_and(v >= lo, v < hi)
        pc = plsc.all_reduce_population_count(m)
        off = cnt_sm[0]
        plsc.store_compressed(nlist.at[pl.ds(off, _L)], v, mask=m)
        plsc.store_compressed(plist.at[pl.ds(off, _L)], iota + i * _L, mask=m)
        cnt_sm[0] = off + pc[0]

    tot = cnt_sm[0]
    nlist[pl.ds(tot, _L)] = jnp.full((_L,), -1, jnp.int32)

    nscan = lax.shift_right_logical(tot + _L - 1, 4)
    kvecs = [iota + (g * _L) for g in range(_EMBED_DIM // _L)]

    s0v = jnp.full((_L,), s0, jnp.int32)
    for r in range(16):
        cnt_sm[8 + r] = 0

    @pl.loop(0, nscan)
    def _cnt(s):
        lv = nlist[pl.ds(s * _L, _L)]
        bv = lax.shift_right_logical(jnp.right_shift(lv, 9) - s0v, 2)
        for r in range(16):
            pc = plsc.all_reduce_population_count(
                bv == jnp.full((_L,), r, jnp.int32))
            cnt_sm[8 + r] = cnt_sm[8 + r] + pc[0]

    run = jnp.int32(0)
    for r in range(16):
        cnt_sm[32 + r] = run
        run = run + cnt_sm[8 + r]
    cnt_sm[32 + 16] = run
    for r in range(16):
        cnt_sm[8 + r] = cnt_sm[32 + r]

    @pl.loop(0, nscan)
    def _sortp(s):
        lv = nlist[pl.ds(s * _L, _L)]
        bv = lax.shift_right_logical(jnp.right_shift(lv, 9) - s0v, 2)
        ev = iota + s * _L
        for r in range(16):
            m_r = bv == jnp.full((_L,), r, jnp.int32)
            off_r = cnt_sm[8 + r]
            plsc.store_compressed(gidx_v.at[pl.ds(off_r, _L)], ev, mask=m_r)
            pc = plsc.all_reduce_population_count(m_r)
            cnt_sm[8 + r] = off_r + pc[0]

    gidx_v[pl.ds(tot, _L)] = jnp.full((_L,), tot, jnp.int32)

    def fire(q, buf, sem):
        @pl.when(q == _NSLAB - 1)
        def _():
            off = pl.multiple_of(q * _W, _G)
            pltpu.async_copy(
                table_hbm.at[:, pl.ds(off, _G)],
                stage.at[buf].at[:, pl.ds(0, _G)],
                sem,
            )

        @pl.when(q < _NSLAB - 1)
        def _():
            off = pl.multiple_of(q * _W, _W)
            pltpu.async_copy(
                table_hbm.at[:, pl.ds(off, _W)], stage.at[buf], sem
            )

    def drain(q, buf, sem):
        @pl.when(q == _NSLAB - 1)
        def _():
            pltpu.make_async_copy(
                table_hbm.at[:, pl.ds(0, _G)],
                stage.at[buf].at[:, pl.ds(0, _G)],
                sem,
            ).wait()

        @pl.when(q < _NSLAB - 1)
        def _():
            pltpu.make_async_copy(
                table_hbm.at[:, pl.ds(0, _W)], stage.at[buf], sem
            ).wait()

    def scan_extract(qcur, buf):
        qv = jnp.full((_L,), qcur, jnp.int32)
        r_idx = lax.shift_right_logical(qcur - s0, 2)
        seg_lo = cnt_sm[32 + r_idx]
        seg_hi = cnt_sm[32 + r_idx + 1]

        @pl.loop(lax.shift_right_logical(seg_lo, 4),
                 lax.shift_right_logical(seg_hi + _L - 1, 4))
        def _s(s):
            ev = gidx_v[pl.ds(s * _L, _L)]
            lv = plsc.load_gather(nlist, [ev])
            m = jnp.right_shift(lv, 9) == qv
            pc = plsc.all_reduce_population_count(m)

            @pl.when(pc[0] > 0)
            def _():
                pv = plsc.load_gather(plist, [ev])
                cv = jnp.bitwise_and(lv, _W - 1)
                mi = jnp.where(m, 1, 0)
                for k in range(_L):
                    @pl.when(mi[k] == 1)
                    def _():
                        c = cv[k]
                        pos = pv[k]
                        rs = cnt_sm[1]
                        slot = jnp.bitwise_and(rs, _CAP - 1)
                        cvec = jnp.full((_L,), c, jnp.int32)
                        for kg, kvec in enumerate(kvecs):
                            vals = plsc.load_gather(stage.at[buf], [kvec, cvec])
                            rows_v[pl.ds(slot * _EMBED_DIM + kg * _L, _L)] = vals
                        pltpu.async_copy(
                            rows_v.at[pl.ds(slot * _EMBED_DIM, _EMBED_DIM)],
                            out_hbm.at[pl.ds(pos * _EMBED_DIM, _EMBED_DIM)],
                            sem_r,
                        )
                        cnt_sm[1] = rs + 1

    fire(s0, 0, sem_a)

    @pl.when(cnts > 1)
    def _():
        fire(s0 + 1, 1, sem_b)

    @pl.loop(0, lax.shift_right_logical(cnts + 1, 1))
    def _pair(u):
        qa = s0 + 2 * u
        drain(qa, 0, sem_a)
        scan_extract(qa, 0)

        @pl.when(2 * u + 2 < cnts)
        def _():
            fire(qa + 2, 0, sem_a)

        @pl.when(2 * u + 1 < cnts)
        def _():
            drain(qa + 1, 1, sem_b)
            scan_extract(qa + 1, 1)

            @pl.when(2 * u + 3 < cnts)
            def _():
                fire(qa + 3, 1, sem_b)

    nrows = cnt_sm[1]

    @pl.loop(0, nrows)
    def _dr(u):
        pltpu.make_async_copy(
            rows_v.at[pl.ds(0, _EMBED_DIM)],
            out_hbm.at[pl.ds(0, _EMBED_DIM)],
            sem_r,
        ).wait()


def kernel(batch, table):
    flat = _gather_kernel(batch, table.T)
    return flat.reshape(_BATCH, _EMBED_DIM)
